# baseline (speedup 1.0000x reference)
import numpy as np

import jax
import jax.numpy as jnp
from jax import lax
from jax.experimental import pallas as pl
from jax.experimental.pallas import tpu as pltpu
from jax.experimental.pallas import tpu_sc as plsc

B, S = 4, 2048
N = B * S
H = 2048
DG = 1024
DQ = 512
EPS = 1e-12

NC, NS = 2, 16
NW = NC * NS
TOK = N // NW
C = 16
NCH = TOK // C
TG = 4
L = 16

_MAGIC = np.int32(0x5F3759DF)
_GDN = lax.GatherDimensionNumbers(offset_dims=(), collapsed_slice_dims=(0,),
                                  start_index_map=(0,))


def _lane_shuffle(x, perm):
    return lax.gather(x, perm[:, None], _GDN, slice_sizes=(1,),
                      mode=lax.GatherScatterMode.PROMISE_IN_BOUNDS)


def _allsum(x):
    lane = lax.iota(jnp.int32, L)
    for k in (8, 4, 2, 1):
        x = x + _lane_shuffle(x, lane ^ k)
    return x


def _rsqrt16(v):
    i = lax.bitcast_convert_type(v, jnp.int32)
    y = lax.bitcast_convert_type(_MAGIC - (i >> 1), jnp.float32)
    for _ in range(3):
        y = y * (1.5 - 0.5 * v * y * y)
    return y


def _sc_body(ids_hbm, tts_hbm, pos_hbm, gate_hbm, qubit_hbm, gamma_hbm,
             beta_hbm, out_hbm, ids_v, tts_v, pos_v, gate_v, q2_v, out_v,
             q01_v, diff_v, gamma_v, beta_v, sem_g, sem_q2, sem_out):
    cid = lax.axis_index("c")
    sid = lax.axis_index("s")
    wid = sid * NC + cid
    wbase = wid * TOK

    pltpu.sync_copy(gamma_hbm, gamma_v)
    pltpu.sync_copy(beta_hbm, beta_v)
    pltpu.sync_copy(ids_hbm.at[pl.ds(wbase, TOK)], ids_v)
    pltpu.sync_copy(tts_hbm.at[pl.ds(wbase, TOK)], tts_v.at[pl.ds(0, TOK)])
    pltpu.sync_copy(pos_hbm.at[pl.ds(wbase, TOK)], pos_v)
    pltpu.sync_copy(qubit_hbm.at[pl.ds(0, 2)], q01_v)

    def qinit(j, carry):
        qa0, qa02, qa1, qa12 = carry
        x0 = q01_v[0, pl.ds(j * L, L)]
        x1 = q01_v[1, pl.ds(j * L, L)]
        diff_v[pl.ds(j * L, L)] = x1 - x0
        return qa0 + x0, qa02 + x0 * x0, qa1 + x1, qa12 + x1 * x1

    zero = jnp.zeros((L,), jnp.float32)
    a0, a02, a1, a12 = plsc.parallel_loop(
        0, DQ // L, unroll=4, carry=(zero, zero, zero, zero))(qinit)
    s0v = _allsum(a0)
    sq0v = _allsum(a02)
    dsv = _allsum(a1) - s0v
    dqv = _allsum(a12) - sq0v
    zidx = jnp.zeros((L,), jnp.int32)

    def issue_gathers(c, b):
        cb = c * C
        pltpu.async_copy(gate_hbm.at[ids_v.at[pl.ds(cb, C)]],
                         gate_v.at[b], sem_g)
        pltpu.async_copy(qubit_hbm.at[pos_v.at[pl.ds(cb, C)]],
                         q2_v.at[b], sem_q2)

    def wait_gathers(b):
        pltpu.make_async_copy(gate_hbm.at[ids_v.at[pl.ds(0, C)]],
                              gate_v.at[b], sem_g).wait()
        pltpu.make_async_copy(qubit_hbm.at[pos_v.at[pl.ds(0, C)]],
                              q2_v.at[b], sem_q2).wait()

    def wait_out(b):
        pltpu.make_async_copy(out_v.at[b], out_hbm.at[pl.ds(wbase, C)],
                              sem_out).wait()

    def compute(c, b):
        for tg in range(C // TG):
            means = []
            rstds = []
            ttfs = []
            for k in range(TG):
                t = tg * TG + k

                def acc_g(j, carry):
                    aa0, aa20, aa1, aa21 = carry
                    x0 = gate_v[b, t, pl.ds((2 * j) * L, L)]
                    x1 = gate_v[b, t, pl.ds((2 * j + 1) * L, L)]
                    return aa0 + x0, aa20 + x0 * x0, aa1 + x1, aa21 + x1 * x1

                def acc_q2(j, carry):
                    aa0, aa20, aa1, aa21 = carry
                    x0 = q2_v[b, t, pl.ds((2 * j) * L, L)]
                    x1 = q2_v[b, t, pl.ds((2 * j + 1) * L, L)]
                    return aa0 + x0, aa20 + x0 * x0, aa1 + x1, aa21 + x1 * x1

                z4 = (zero, zero, zero, zero)
                g0, g20, g1, g21 = plsc.parallel_loop(
                    0, DG // (2 * L), unroll=4, carry=z4)(acc_g)
                g0, g20, g1, g21 = plsc.parallel_loop(
                    0, DQ // (2 * L), unroll=4,
                    carry=(g0, g20, g1, g21))(acc_q2)
                ttv = tts_v[pl.ds(c * C + t, L)]
                ttf = _lane_shuffle(ttv.astype(jnp.float32), zidx)
                s = _allsum(g0 + g1) + s0v + ttf * dsv
                s2 = _allsum(g20 + g21) + sq0v + ttf * dqv
                mean_v = s * (1.0 / H)
                var_v = s2 * (1.0 / H) - mean_v * mean_v
                means.append(mean_v)
                rstds.append(_rsqrt16(var_v + EPS))
                ttfs.append(ttf)

            def norm_g(j):
                g = gamma_v[pl.ds(j * L, L)]
                bta = beta_v[pl.ds(j * L, L)]
                for k in range(TG):
                    t = tg * TG + k
                    x = gate_v[b, t, pl.ds(j * L, L)]
                    out_v[b, t, pl.ds(j * L, L)] = \
                        (x - means[k]) * rstds[k] * g + bta

            def norm_q1(j):
                g = gamma_v[pl.ds(DG + j * L, L)]
                bta = beta_v[pl.ds(DG + j * L, L)]
                r0 = q01_v[0, pl.ds(j * L, L)]
                dd = diff_v[pl.ds(j * L, L)]
                for k in range(TG):
                    t = tg * TG + k
                    x = r0 + ttfs[k] * dd
                    out_v[b, t, pl.ds(DG + j * L, L)] = \
                        (x - means[k]) * rstds[k] * g + bta

            def norm_q2(j):
                g = gamma_v[pl.ds(DG + DQ + j * L, L)]
                bta = beta_v[pl.ds(DG + DQ + j * L, L)]
                for k in range(TG):
                    t = tg * TG + k
                    x = q2_v[b, t, pl.ds(j * L, L)]
                    out_v[b, t, pl.ds(DG + DQ + j * L, L)] = \
                        (x - means[k]) * rstds[k] * g + bta

            plsc.parallel_loop(0, DG // L, unroll=4)(norm_g)
            plsc.parallel_loop(0, DQ // L, unroll=4)(norm_q1)
            plsc.parallel_loop(0, DQ // L, unroll=4)(norm_q2)

    issue_gathers(0, 0)

    def pair_body(i, _):
        c0 = 2 * i

        issue_gathers(c0 + 1, 1)
        wait_gathers(0)

        @pl.when(i >= 1)
        def _():
            wait_out(0)

        compute(c0, 0)
        pltpu.async_copy(out_v.at[0], out_hbm.at[pl.ds(wbase + c0 * C, C)],
                         sem_out)

        @pl.when(i + 1 < NCH // 2)
        def _():
            issue_gathers(c0 + 2, 0)

        wait_gathers(1)

        @pl.when(i >= 1)
        def _():
            wait_out(1)

        compute(c0 + 1, 1)
        pltpu.async_copy(out_v.at[1],
                         out_hbm.at[pl.ds(wbase + (c0 + 1) * C, C)], sem_out)
        return 0

    lax.fori_loop(0, NCH // 2, pair_body, 0)
    wait_out(0)
    wait_out(1)


@jax.jit
def kernel(input_ids, token_type_ids, position_ids, gate_table, qubit_table,
           ln_gamma, ln_beta):
    ids = input_ids.astype(jnp.int32).reshape(N)
    tts = token_type_ids.astype(jnp.int32).reshape(N)
    pos = position_ids.astype(jnp.int32).reshape(N)

    mesh = plsc.VectorSubcoreMesh(core_axis_name="c", subcore_axis_name="s",
                                  num_cores=NC, num_subcores=NS)
    run = pl.kernel(
        _sc_body,
        out_type=jax.ShapeDtypeStruct((N, H), jnp.float32),
        mesh=mesh,
        scratch_types=[
            pltpu.VMEM((TOK,), jnp.int32),
            pltpu.VMEM((TOK + L,), jnp.int32),
            pltpu.VMEM((TOK,), jnp.int32),
            pltpu.VMEM((2, C, DG), jnp.float32),
            pltpu.VMEM((2, C, DQ), jnp.float32),
            pltpu.VMEM((2, C, H), jnp.float32),
            pltpu.VMEM((2, DQ), jnp.float32),
            pltpu.VMEM((DQ,), jnp.float32),
            pltpu.VMEM((H,), jnp.float32),
            pltpu.VMEM((H,), jnp.float32),
            pltpu.SemaphoreType.DMA,
            pltpu.SemaphoreType.DMA,
            pltpu.SemaphoreType.DMA,
        ],
    )
    out = run(ids, tts, pos, gate_table, qubit_table, ln_gamma, ln_beta)
    return out.reshape(B, S, H)


# speedup vs baseline: 1.0623x; 1.0623x over previous
import numpy as np

import jax
import jax.numpy as jnp
from jax import lax
from jax.experimental import pallas as pl
from jax.experimental.pallas import tpu as pltpu
from jax.experimental.pallas import tpu_sc as plsc

B, S = 4, 2048
N = B * S
H = 2048
DG = 1024
DQ = 512
EPS = 1e-12

NC, NS = 2, 16
NW = NC * NS
TOK = N // NW
C = 8
NCH = TOK // C
TG = 4
L = 16

_MAGIC = np.int32(0x5F3759DF)
_GDN = lax.GatherDimensionNumbers(offset_dims=(), collapsed_slice_dims=(0,),
                                  start_index_map=(0,))


def _lane_shuffle(x, perm):
    return lax.gather(x, perm[:, None], _GDN, slice_sizes=(1,),
                      mode=lax.GatherScatterMode.PROMISE_IN_BOUNDS)


def _allsum(x):
    lane = lax.iota(jnp.int32, L)
    for k in (8, 4, 2, 1):
        x = x + _lane_shuffle(x, lane ^ k)
    return x


def _rsqrt16(v):
    i = lax.bitcast_convert_type(v, jnp.int32)
    y = lax.bitcast_convert_type(_MAGIC - (i >> 1), jnp.float32)
    for _ in range(3):
        y = y * (1.5 - 0.5 * v * y * y)
    return y


def _sc_body(ids_hbm, tts_hbm, pos_hbm, gate_hbm, qubit_hbm, gamma_hbm,
             beta_hbm, out_hbm, ids_v, tts_v, pos_v, gate_v, q2_v, out_v,
             q01_v, diff_v, gamma_v, beta_v, sem_g, sem_q2, sem_out):
    cid = lax.axis_index("c")
    sid = lax.axis_index("s")
    wid = sid * NC + cid
    wbase = wid * TOK

    pltpu.sync_copy(gamma_hbm, gamma_v)
    pltpu.sync_copy(beta_hbm, beta_v)
    pltpu.sync_copy(ids_hbm.at[pl.ds(wbase, TOK)], ids_v)
    pltpu.sync_copy(tts_hbm.at[pl.ds(wbase, TOK)], tts_v.at[pl.ds(0, TOK)])
    pltpu.sync_copy(pos_hbm.at[pl.ds(wbase, TOK)], pos_v)
    pltpu.sync_copy(qubit_hbm.at[pl.ds(0, 2)], q01_v)

    def qinit(j, carry):
        qa0, qa02, qa1, qa12 = carry
        x0 = q01_v[0, pl.ds(j * L, L)]
        x1 = q01_v[1, pl.ds(j * L, L)]
        diff_v[pl.ds(j * L, L)] = x1 - x0
        return qa0 + x0, qa02 + x0 * x0, qa1 + x1, qa12 + x1 * x1

    zero = jnp.zeros((L,), jnp.float32)
    a0, a02, a1, a12 = plsc.parallel_loop(
        0, DQ // L, unroll=4, carry=(zero, zero, zero, zero))(qinit)
    s0v = _allsum(a0)
    sq0v = _allsum(a02)
    dsv = _allsum(a1) - s0v
    dqv = _allsum(a12) - sq0v
    zidx = jnp.zeros((L,), jnp.int32)

    def issue_gathers(c, b):
        cb = c * C
        pltpu.async_copy(gate_hbm.at[ids_v.at[pl.ds(cb, C)]],
                         gate_v.at[b], sem_g)
        pltpu.async_copy(qubit_hbm.at[pos_v.at[pl.ds(cb, C)]],
                         q2_v.at[b], sem_q2)

    def wait_gathers(b):
        pltpu.make_async_copy(gate_hbm.at[ids_v.at[pl.ds(0, C)]],
                              gate_v.at[b], sem_g).wait()
        pltpu.make_async_copy(qubit_hbm.at[pos_v.at[pl.ds(0, C)]],
                              q2_v.at[b], sem_q2).wait()

    def wait_out(b):
        pltpu.make_async_copy(out_v.at[b], out_hbm.at[pl.ds(wbase, C)],
                              sem_out).wait()

    def compute(c, b):
        for tg in range(C // TG):
            means = []
            rstds = []
            ttfs = []
            for k in range(TG):
                t = tg * TG + k

                def acc_g(j, carry):
                    aa0, aa20, aa1, aa21 = carry
                    x0 = gate_v[b, t, pl.ds((2 * j) * L, L)]
                    x1 = gate_v[b, t, pl.ds((2 * j + 1) * L, L)]
                    return aa0 + x0, aa20 + x0 * x0, aa1 + x1, aa21 + x1 * x1

                def acc_q2(j, carry):
                    aa0, aa20, aa1, aa21 = carry
                    x0 = q2_v[b, t, pl.ds((2 * j) * L, L)]
                    x1 = q2_v[b, t, pl.ds((2 * j + 1) * L, L)]
                    return aa0 + x0, aa20 + x0 * x0, aa1 + x1, aa21 + x1 * x1

                z4 = (zero, zero, zero, zero)
                g0, g20, g1, g21 = plsc.parallel_loop(
                    0, DG // (2 * L), unroll=4, carry=z4)(acc_g)
                g0, g20, g1, g21 = plsc.parallel_loop(
                    0, DQ // (2 * L), unroll=4,
                    carry=(g0, g20, g1, g21))(acc_q2)
                ttv = tts_v[pl.ds(c * C + t, L)]
                ttf = _lane_shuffle(ttv.astype(jnp.float32), zidx)
                s = _allsum(g0 + g1) + s0v + ttf * dsv
                s2 = _allsum(g20 + g21) + sq0v + ttf * dqv
                mean_v = s * (1.0 / H)
                var_v = s2 * (1.0 / H) - mean_v * mean_v
                means.append(mean_v)
                rstds.append(_rsqrt16(var_v + EPS))
                ttfs.append(ttf)

            def norm_g(j):
                g = gamma_v[pl.ds(j * L, L)]
                bta = beta_v[pl.ds(j * L, L)]
                for k in range(TG):
                    t = tg * TG + k
                    x = gate_v[b, t, pl.ds(j * L, L)]
                    out_v[b, t, pl.ds(j * L, L)] = \
                        (x - means[k]) * rstds[k] * g + bta

            def norm_q1(j):
                g = gamma_v[pl.ds(DG + j * L, L)]
                bta = beta_v[pl.ds(DG + j * L, L)]
                r0 = q01_v[0, pl.ds(j * L, L)]
                dd = diff_v[pl.ds(j * L, L)]
                for k in range(TG):
                    t = tg * TG + k
                    x = r0 + ttfs[k] * dd
                    out_v[b, t, pl.ds(DG + j * L, L)] = \
                        (x - means[k]) * rstds[k] * g + bta

            def norm_q2(j):
                g = gamma_v[pl.ds(DG + DQ + j * L, L)]
                bta = beta_v[pl.ds(DG + DQ + j * L, L)]
                for k in range(TG):
                    t = tg * TG + k
                    x = q2_v[b, t, pl.ds(j * L, L)]
                    out_v[b, t, pl.ds(DG + DQ + j * L, L)] = \
                        (x - means[k]) * rstds[k] * g + bta

            plsc.parallel_loop(0, DG // L, unroll=4)(norm_g)
            plsc.parallel_loop(0, DQ // L, unroll=4)(norm_q1)
            plsc.parallel_loop(0, DQ // L, unroll=4)(norm_q2)

    issue_gathers(0, 0)

    def pair_body(i, _):
        c0 = 2 * i

        issue_gathers(c0 + 1, 1)
        wait_gathers(0)

        @pl.when(i >= 1)
        def _():
            wait_out(0)

        compute(c0, 0)
        pltpu.async_copy(out_v.at[0], out_hbm.at[pl.ds(wbase + c0 * C, C)],
                         sem_out)

        @pl.when(i + 1 < NCH // 2)
        def _():
            issue_gathers(c0 + 2, 0)

        wait_gathers(1)

        @pl.when(i >= 1)
        def _():
            wait_out(1)

        compute(c0 + 1, 1)
        pltpu.async_copy(out_v.at[1],
                         out_hbm.at[pl.ds(wbase + (c0 + 1) * C, C)], sem_out)
        return 0

    lax.fori_loop(0, NCH // 2, pair_body, 0)
    wait_out(0)
    wait_out(1)


@jax.jit
def kernel(input_ids, token_type_ids, position_ids, gate_table, qubit_table,
           ln_gamma, ln_beta):
    ids = input_ids.astype(jnp.int32).reshape(N)
    tts = token_type_ids.astype(jnp.int32).reshape(N)
    pos = position_ids.astype(jnp.int32).reshape(N)

    mesh = plsc.VectorSubcoreMesh(core_axis_name="c", subcore_axis_name="s",
                                  num_cores=NC, num_subcores=NS)
    run = pl.kernel(
        _sc_body,
        out_type=jax.ShapeDtypeStruct((N, H), jnp.float32),
        mesh=mesh,
        scratch_types=[
            pltpu.VMEM((TOK,), jnp.int32),
            pltpu.VMEM((TOK + L,), jnp.int32),
            pltpu.VMEM((TOK,), jnp.int32),
            pltpu.VMEM((2, C, DG), jnp.float32),
            pltpu.VMEM((2, C, DQ), jnp.float32),
            pltpu.VMEM((2, C, H), jnp.float32),
            pltpu.VMEM((2, DQ), jnp.float32),
            pltpu.VMEM((DQ,), jnp.float32),
            pltpu.VMEM((H,), jnp.float32),
            pltpu.VMEM((H,), jnp.float32),
            pltpu.SemaphoreType.DMA,
            pltpu.SemaphoreType.DMA,
            pltpu.SemaphoreType.DMA,
        ],
    )
    out = run(ids, tts, pos, gate_table, qubit_table, ln_gamma, ln_beta)
    return out.reshape(B, S, H)


# speedup vs baseline: 1.1259x; 1.0599x over previous
import numpy as np

import jax
import jax.numpy as jnp
from jax import lax
from jax.experimental import pallas as pl
from jax.experimental.pallas import tpu as pltpu
from jax.experimental.pallas import tpu_sc as plsc

B, S = 4, 2048
N = B * S
H = 2048
DG = 1024
DQ = 512
EPS = 1e-12

NC, NS = 2, 16
NW = NC * NS
TOK = N // NW
C = 8
NCH = TOK // C
TG = 4
L = 16

_MAGIC = np.int32(0x5F3759DF)
_GDN = lax.GatherDimensionNumbers(offset_dims=(), collapsed_slice_dims=(0,),
                                  start_index_map=(0,))


def _lane_shuffle(x, perm):
    return lax.gather(x, perm[:, None], _GDN, slice_sizes=(1,),
                      mode=lax.GatherScatterMode.PROMISE_IN_BOUNDS)


def _allsum(x):
    lane = lax.iota(jnp.int32, L)
    for k in (8, 4, 2, 1):
        x = x + _lane_shuffle(x, lane ^ k)
    return x


def _rsqrt16(v):
    i = lax.bitcast_convert_type(v, jnp.int32)
    y = lax.bitcast_convert_type(_MAGIC - (i >> 1), jnp.float32)
    for _ in range(3):
        y = y * (1.5 - 0.5 * v * y * y)
    return y


def _sc_body(ids_hbm, tts_hbm, pos_hbm, gate_hbm, qubit_hbm, gamma_hbm,
             beta_hbm, out_hbm, ids_v, tts_v, pos_v, gate_v, q2_v, out_v,
             q01_v, diff_v, gamma_v, beta_v, sem_g, sem_q2, sem_out):
    cid = lax.axis_index("c")
    sid = lax.axis_index("s")
    wid = sid * NC + cid
    wbase = wid * TOK

    pltpu.sync_copy(gamma_hbm, gamma_v)
    pltpu.sync_copy(beta_hbm, beta_v)
    pltpu.sync_copy(ids_hbm.at[pl.ds(wbase, TOK)], ids_v)
    pltpu.sync_copy(tts_hbm.at[pl.ds(wbase, TOK)], tts_v.at[pl.ds(0, TOK)])
    pltpu.sync_copy(pos_hbm.at[pl.ds(wbase, TOK)], pos_v)
    pltpu.sync_copy(qubit_hbm.at[pl.ds(0, 2)], q01_v)

    def qinit(j, carry):
        qa0, qa02, qa1, qa12 = carry
        x0 = q01_v[0, pl.ds(j * L, L)]
        x1 = q01_v[1, pl.ds(j * L, L)]
        diff_v[pl.ds(j * L, L)] = x1 - x0
        return qa0 + x0, qa02 + x0 * x0, qa1 + x1, qa12 + x1 * x1

    zero = jnp.zeros((L,), jnp.float32)
    a0, a02, a1, a12 = plsc.parallel_loop(
        0, DQ // L, unroll=4, carry=(zero, zero, zero, zero))(qinit)
    s0v = _allsum(a0)
    sq0v = _allsum(a02)
    dsv = _allsum(a1) - s0v
    dqv = _allsum(a12) - sq0v
    zidx = jnp.zeros((L,), jnp.int32)

    def issue_gathers(c, b):
        cb = c * C
        pltpu.async_copy(gate_hbm.at[ids_v.at[pl.ds(cb, C)]],
                         gate_v.at[b], sem_g)
        pltpu.async_copy(qubit_hbm.at[pos_v.at[pl.ds(cb, C)]],
                         q2_v.at[b], sem_q2)

    def wait_gathers(b):
        pltpu.make_async_copy(gate_hbm.at[ids_v.at[pl.ds(0, C)]],
                              gate_v.at[b], sem_g).wait()
        pltpu.make_async_copy(qubit_hbm.at[pos_v.at[pl.ds(0, C)]],
                              q2_v.at[b], sem_q2).wait()

    def wait_out(b):
        pltpu.make_async_copy(out_v.at[b], out_hbm.at[pl.ds(wbase, C)],
                              sem_out).wait()

    def compute(c, b):
        for tg in range(C // TG):
            means = []
            rstds = []
            ttfs = []
            for k in range(TG):
                t = tg * TG + k

                def acc_g(j, carry):
                    aa0, aa20, aa1, aa21 = carry
                    x0 = gate_v[b, t, pl.ds((2 * j) * L, L)]
                    x1 = gate_v[b, t, pl.ds((2 * j + 1) * L, L)]
                    return aa0 + x0, aa20 + x0 * x0, aa1 + x1, aa21 + x1 * x1

                def acc_q2(j, carry):
                    aa0, aa20, aa1, aa21 = carry
                    x0 = q2_v[b, t, pl.ds((2 * j) * L, L)]
                    x1 = q2_v[b, t, pl.ds((2 * j + 1) * L, L)]
                    return aa0 + x0, aa20 + x0 * x0, aa1 + x1, aa21 + x1 * x1

                z4 = (zero, zero, zero, zero)
                g0, g20, g1, g21 = plsc.parallel_loop(
                    0, DG // (2 * L), unroll=4, carry=z4)(acc_g)
                g0, g20, g1, g21 = plsc.parallel_loop(
                    0, DQ // (2 * L), unroll=4,
                    carry=(g0, g20, g1, g21))(acc_q2)
                ttv = tts_v[pl.ds(c * C + t, L)]
                ttf = _lane_shuffle(ttv.astype(jnp.float32), zidx)
                s = _allsum(g0 + g1) + s0v + ttf * dsv
                s2 = _allsum(g20 + g21) + sq0v + ttf * dqv
                mean_v = s * (1.0 / H)
                var_v = s2 * (1.0 / H) - mean_v * mean_v
                means.append(mean_v)
                rstds.append(_rsqrt16(var_v + EPS))
                ttfs.append(ttf)

            def norm_g(j):
                g = gamma_v[pl.ds(j * L, L)]
                bta = beta_v[pl.ds(j * L, L)]
                for k in range(TG):
                    t = tg * TG + k
                    x = gate_v[b, t, pl.ds(j * L, L)]
                    out_v[b, t, pl.ds(j * L, L)] = \
                        (x - means[k]) * rstds[k] * g + bta

            def norm_q1(j):
                g = gamma_v[pl.ds(DG + j * L, L)]
                bta = beta_v[pl.ds(DG + j * L, L)]
                r0 = q01_v[0, pl.ds(j * L, L)]
                dd = diff_v[pl.ds(j * L, L)]
                for k in range(TG):
                    t = tg * TG + k
                    x = r0 + ttfs[k] * dd
                    out_v[b, t, pl.ds(DG + j * L, L)] = \
                        (x - means[k]) * rstds[k] * g + bta

            def norm_q2(j):
                g = gamma_v[pl.ds(DG + DQ + j * L, L)]
                bta = beta_v[pl.ds(DG + DQ + j * L, L)]
                for k in range(TG):
                    t = tg * TG + k
                    x = q2_v[b, t, pl.ds(j * L, L)]
                    out_v[b, t, pl.ds(DG + DQ + j * L, L)] = \
                        (x - means[k]) * rstds[k] * g + bta

            plsc.parallel_loop(0, DG // L, unroll=2)(norm_g)
            plsc.parallel_loop(0, DQ // L, unroll=2)(norm_q1)
            plsc.parallel_loop(0, DQ // L, unroll=2)(norm_q2)

    issue_gathers(0, 0)

    def pair_body(i, _):
        c0 = 2 * i

        issue_gathers(c0 + 1, 1)
        wait_gathers(0)

        @pl.when(i >= 1)
        def _():
            wait_out(0)

        compute(c0, 0)
        pltpu.async_copy(out_v.at[0], out_hbm.at[pl.ds(wbase + c0 * C, C)],
                         sem_out)

        @pl.when(i + 1 < NCH // 2)
        def _():
            issue_gathers(c0 + 2, 0)

        wait_gathers(1)

        @pl.when(i >= 1)
        def _():
            wait_out(1)

        compute(c0 + 1, 1)
        pltpu.async_copy(out_v.at[1],
                         out_hbm.at[pl.ds(wbase + (c0 + 1) * C, C)], sem_out)
        return 0

    lax.fori_loop(0, NCH // 2, pair_body, 0)
    wait_out(0)
    wait_out(1)


@jax.jit
def kernel(input_ids, token_type_ids, position_ids, gate_table, qubit_table,
           ln_gamma, ln_beta):
    ids = input_ids.astype(jnp.int32).reshape(N)
    tts = token_type_ids.astype(jnp.int32).reshape(N)
    pos = position_ids.astype(jnp.int32).reshape(N)

    mesh = plsc.VectorSubcoreMesh(core_axis_name="c", subcore_axis_name="s",
                                  num_cores=NC, num_subcores=NS)
    run = pl.kernel(
        _sc_body,
        out_type=jax.ShapeDtypeStruct((N, H), jnp.float32),
        mesh=mesh,
        scratch_types=[
            pltpu.VMEM((TOK,), jnp.int32),
            pltpu.VMEM((TOK + L,), jnp.int32),
            pltpu.VMEM((TOK,), jnp.int32),
            pltpu.VMEM((2, C, DG), jnp.float32),
            pltpu.VMEM((2, C, DQ), jnp.float32),
            pltpu.VMEM((2, C, H), jnp.float32),
            pltpu.VMEM((2, DQ), jnp.float32),
            pltpu.VMEM((DQ,), jnp.float32),
            pltpu.VMEM((H,), jnp.float32),
            pltpu.VMEM((H,), jnp.float32),
            pltpu.SemaphoreType.DMA,
            pltpu.SemaphoreType.DMA,
            pltpu.SemaphoreType.DMA,
        ],
    )
    out = run(ids, tts, pos, gate_table, qubit_table, ln_gamma, ln_beta)
    return out.reshape(B, S, H)


# speedup vs baseline: 1.3507x; 1.1997x over previous
import numpy as np

import jax
import jax.numpy as jnp
from jax import lax
from jax.experimental import pallas as pl
from jax.experimental.pallas import tpu as pltpu
from jax.experimental.pallas import tpu_sc as plsc

B, S = 4, 2048
N = B * S
H = 2048
DG = 1024
DQ = 512
EPS = 1e-12

NC, NS = 2, 16
NW = NC * NS
TOK = N // NW
C = 8
NCH = TOK // C
TG = 4
L = 16

_MAGIC = np.int32(0x5F3759DF)
_GDN = lax.GatherDimensionNumbers(offset_dims=(), collapsed_slice_dims=(0,),
                                  start_index_map=(0,))


def _lane_shuffle(x, perm):
    return lax.gather(x, perm[:, None], _GDN, slice_sizes=(1,),
                      mode=lax.GatherScatterMode.PROMISE_IN_BOUNDS)


def _allsum(x):
    lane = lax.iota(jnp.int32, L)
    for k in (8, 4, 2, 1):
        x = x + _lane_shuffle(x, lane ^ k)
    return x


def _rsqrt16(v):
    i = lax.bitcast_convert_type(v, jnp.int32)
    y = lax.bitcast_convert_type(_MAGIC - (i >> 1), jnp.float32)
    for _ in range(3):
        y = y * (1.5 - 0.5 * v * y * y)
    return y


def _sc_body(ids_hbm, tts_hbm, pos_hbm, gate_hbm, qubit_hbm, gamma_hbm,
             beta_hbm, out_hbm, ids_v, tts_v, pos_v, gate_v, q2_v, out_v,
             q01_v, diff_v, gamma_v, beta_v, sem_g, sem_q2, sem_out):
    cid = lax.axis_index("c")
    sid = lax.axis_index("s")
    wid = sid * NC + cid
    wbase = wid * TOK

    pltpu.sync_copy(gamma_hbm, gamma_v)
    pltpu.sync_copy(beta_hbm, beta_v)
    pltpu.sync_copy(ids_hbm.at[pl.ds(wbase, TOK)], ids_v)
    pltpu.sync_copy(tts_hbm.at[pl.ds(wbase, TOK)], tts_v.at[pl.ds(0, TOK)])
    pltpu.sync_copy(pos_hbm.at[pl.ds(wbase, TOK)], pos_v)
    pltpu.sync_copy(qubit_hbm.at[pl.ds(0, 2)], q01_v)

    def qinit(j, carry):
        qa0, qa02, qa1, qa12 = carry
        x0 = q01_v[0, pl.ds(j * L, L)]
        x1 = q01_v[1, pl.ds(j * L, L)]
        diff_v[pl.ds(j * L, L)] = x1 - x0
        return qa0 + x0, qa02 + x0 * x0, qa1 + x1, qa12 + x1 * x1

    zero = jnp.zeros((L,), jnp.float32)
    a0, a02, a1, a12 = plsc.parallel_loop(
        0, DQ // L, unroll=4, carry=(zero, zero, zero, zero))(qinit)
    s0v = _allsum(a0)
    sq0v = _allsum(a02)
    dsv = _allsum(a1) - s0v
    dqv = _allsum(a12) - sq0v
    zidx = jnp.zeros((L,), jnp.int32)

    def issue_gathers(c, b):
        cb = c * C
        pltpu.async_copy(gate_hbm.at[ids_v.at[pl.ds(cb, C)]],
                         gate_v.at[b], sem_g)
        pltpu.async_copy(qubit_hbm.at[pos_v.at[pl.ds(cb, C)]],
                         q2_v.at[b], sem_q2)

    def wait_gathers(b):
        pltpu.make_async_copy(gate_hbm.at[ids_v.at[pl.ds(0, C)]],
                              gate_v.at[b], sem_g).wait()
        pltpu.make_async_copy(qubit_hbm.at[pos_v.at[pl.ds(0, C)]],
                              q2_v.at[b], sem_q2).wait()

    def wait_out(b):
        pltpu.make_async_copy(out_v.at[b], out_hbm.at[pl.ds(wbase, C)],
                              sem_out).wait()

    def compute(c, b):
        for tg in range(C // TG):
            means = []
            rstds = []
            ttfs = []

            def acc_g(j, carry):
                outs = []
                for k in range(TG):
                    t = tg * TG + k
                    x = gate_v[b, t, pl.ds(j * L, L)]
                    outs += [carry[2 * k] + x, carry[2 * k + 1] + x * x]
                return tuple(outs)

            def acc_q2(j, carry):
                outs = []
                for k in range(TG):
                    t = tg * TG + k
                    x = q2_v[b, t, pl.ds(j * L, L)]
                    outs += [carry[2 * k] + x, carry[2 * k + 1] + x * x]
                return tuple(outs)

            carr = (zero,) * (2 * TG)
            carr = plsc.parallel_loop(0, DG // L, unroll=2,
                                      carry=carr)(acc_g)
            carr = plsc.parallel_loop(0, DQ // L, unroll=2,
                                      carry=carr)(acc_q2)
            for k in range(TG):
                t = tg * TG + k
                ttv = tts_v[pl.ds(c * C + t, L)]
                ttf = _lane_shuffle(ttv.astype(jnp.float32), zidx)
                s = _allsum(carr[2 * k]) + s0v + ttf * dsv
                s2 = _allsum(carr[2 * k + 1]) + sq0v + ttf * dqv
                mean_v = s * (1.0 / H)
                var_v = s2 * (1.0 / H) - mean_v * mean_v
                means.append(mean_v)
                rstds.append(_rsqrt16(var_v + EPS))
                ttfs.append(ttf)

            def norm_g(j):
                g = gamma_v[pl.ds(j * L, L)]
                bta = beta_v[pl.ds(j * L, L)]
                for k in range(TG):
                    t = tg * TG + k
                    x = gate_v[b, t, pl.ds(j * L, L)]
                    out_v[b, t, pl.ds(j * L, L)] = \
                        (x - means[k]) * rstds[k] * g + bta

            def norm_q1(j):
                g = gamma_v[pl.ds(DG + j * L, L)]
                bta = beta_v[pl.ds(DG + j * L, L)]
                r0 = q01_v[0, pl.ds(j * L, L)]
                dd = diff_v[pl.ds(j * L, L)]
                for k in range(TG):
                    t = tg * TG + k
                    x = r0 + ttfs[k] * dd
                    out_v[b, t, pl.ds(DG + j * L, L)] = \
                        (x - means[k]) * rstds[k] * g + bta

            def norm_q2(j):
                g = gamma_v[pl.ds(DG + DQ + j * L, L)]
                bta = beta_v[pl.ds(DG + DQ + j * L, L)]
                for k in range(TG):
                    t = tg * TG + k
                    x = q2_v[b, t, pl.ds(j * L, L)]
                    out_v[b, t, pl.ds(DG + DQ + j * L, L)] = \
                        (x - means[k]) * rstds[k] * g + bta

            plsc.parallel_loop(0, DG // L, unroll=2)(norm_g)
            plsc.parallel_loop(0, DQ // L, unroll=2)(norm_q1)
            plsc.parallel_loop(0, DQ // L, unroll=2)(norm_q2)

    issue_gathers(0, 0)

    def pair_body(i, _):
        c0 = 2 * i

        issue_gathers(c0 + 1, 1)
        wait_gathers(0)

        @pl.when(i >= 1)
        def _():
            wait_out(0)

        compute(c0, 0)
        pltpu.async_copy(out_v.at[0], out_hbm.at[pl.ds(wbase + c0 * C, C)],
                         sem_out)

        @pl.when(i + 1 < NCH // 2)
        def _():
            issue_gathers(c0 + 2, 0)

        wait_gathers(1)

        @pl.when(i >= 1)
        def _():
            wait_out(1)

        compute(c0 + 1, 1)
        pltpu.async_copy(out_v.at[1],
                         out_hbm.at[pl.ds(wbase + (c0 + 1) * C, C)], sem_out)
        return 0

    lax.fori_loop(0, NCH // 2, pair_body, 0)
    wait_out(0)
    wait_out(1)


@jax.jit
def kernel(input_ids, token_type_ids, position_ids, gate_table, qubit_table,
           ln_gamma, ln_beta):
    ids = input_ids.astype(jnp.int32).reshape(N)
    tts = token_type_ids.astype(jnp.int32).reshape(N)
    pos = position_ids.astype(jnp.int32).reshape(N)

    mesh = plsc.VectorSubcoreMesh(core_axis_name="c", subcore_axis_name="s",
                                  num_cores=NC, num_subcores=NS)
    run = pl.kernel(
        _sc_body,
        out_type=jax.ShapeDtypeStruct((N, H), jnp.float32),
        mesh=mesh,
        scratch_types=[
            pltpu.VMEM((TOK,), jnp.int32),
            pltpu.VMEM((TOK + L,), jnp.int32),
            pltpu.VMEM((TOK,), jnp.int32),
            pltpu.VMEM((2, C, DG), jnp.float32),
            pltpu.VMEM((2, C, DQ), jnp.float32),
            pltpu.VMEM((2, C, H), jnp.float32),
            pltpu.VMEM((2, DQ), jnp.float32),
            pltpu.VMEM((DQ,), jnp.float32),
            pltpu.VMEM((H,), jnp.float32),
            pltpu.VMEM((H,), jnp.float32),
            pltpu.SemaphoreType.DMA,
            pltpu.SemaphoreType.DMA,
            pltpu.SemaphoreType.DMA,
        ],
    )
    out = run(ids, tts, pos, gate_table, qubit_table, ln_gamma, ln_beta)
    return out.reshape(B, S, H)


# speedup vs baseline: 1.3518x; 1.0008x over previous
import numpy as np

import jax
import jax.numpy as jnp
from jax import lax
from jax.experimental import pallas as pl
from jax.experimental.pallas import tpu as pltpu
from jax.experimental.pallas import tpu_sc as plsc

B, S = 4, 2048
N = B * S
H = 2048
DG = 1024
DQ = 512
EPS = 1e-12

NC, NS = 2, 16
NW = NC * NS
TOK = N // NW
C = 8
NCH = TOK // C
TG = 4
L = 16

_MAGIC = np.int32(0x5F3759DF)
_GDN = lax.GatherDimensionNumbers(offset_dims=(), collapsed_slice_dims=(0,),
                                  start_index_map=(0,))


def _lane_shuffle(x, perm):
    return lax.gather(x, perm[:, None], _GDN, slice_sizes=(1,),
                      mode=lax.GatherScatterMode.PROMISE_IN_BOUNDS)


def _allsum(x):
    lane = lax.iota(jnp.int32, L)
    for k in (8, 4, 2, 1):
        x = x + _lane_shuffle(x, lane ^ k)
    return x


def _rsqrt16(v):
    i = lax.bitcast_convert_type(v, jnp.int32)
    y = lax.bitcast_convert_type(_MAGIC - (i >> 1), jnp.float32)
    for _ in range(3):
        y = y * (1.5 - 0.5 * v * y * y)
    return y


def _sc_body(ids_hbm, tts_hbm, pos_hbm, gate_hbm, qubit_hbm, gamma_hbm,
             beta_hbm, out_hbm, ids_v, tts_v, pos_v, gate_v, q2_v, out_v,
             q01_v, diff_v, gamma_v, beta_v, sem_g, sem_q2, sem_out):
    cid = lax.axis_index("c")
    sid = lax.axis_index("s")
    wid = sid * NC + cid
    wbase = wid * TOK

    pltpu.sync_copy(gamma_hbm, gamma_v)
    pltpu.sync_copy(beta_hbm, beta_v)
    pltpu.sync_copy(ids_hbm.at[pl.ds(wbase, TOK)], ids_v)
    pltpu.sync_copy(tts_hbm.at[pl.ds(wbase, TOK)], tts_v.at[pl.ds(0, TOK)])
    pltpu.sync_copy(pos_hbm.at[pl.ds(wbase, TOK)], pos_v)
    pltpu.sync_copy(qubit_hbm.at[pl.ds(0, 2)], q01_v)

    def qinit(j, carry):
        qa0, qa02, qa1, qa12 = carry
        x0 = q01_v[0, pl.ds(j * L, L)]
        x1 = q01_v[1, pl.ds(j * L, L)]
        diff_v[pl.ds(j * L, L)] = x1 - x0
        return qa0 + x0, qa02 + x0 * x0, qa1 + x1, qa12 + x1 * x1

    zero = jnp.zeros((L,), jnp.float32)
    a0, a02, a1, a12 = plsc.parallel_loop(
        0, DQ // L, unroll=4, carry=(zero, zero, zero, zero))(qinit)
    s0v = _allsum(a0)
    sq0v = _allsum(a02)
    dsv = _allsum(a1) - s0v
    dqv = _allsum(a12) - sq0v
    zidx = jnp.zeros((L,), jnp.int32)

    def issue_gathers(c, b):
        cb = c * C
        pltpu.async_copy(gate_hbm.at[ids_v.at[pl.ds(cb, C)]],
                         gate_v.at[b], sem_g)
        pltpu.async_copy(qubit_hbm.at[pos_v.at[pl.ds(cb, C)]],
                         q2_v.at[b], sem_q2)

    def wait_gathers(b):
        pltpu.make_async_copy(gate_hbm.at[ids_v.at[pl.ds(0, C)]],
                              gate_v.at[b], sem_g).wait()
        pltpu.make_async_copy(qubit_hbm.at[pos_v.at[pl.ds(0, C)]],
                              q2_v.at[b], sem_q2).wait()

    def wait_out(b):
        pltpu.make_async_copy(out_v.at[b], out_hbm.at[pl.ds(wbase, C)],
                              sem_out).wait()

    def compute(c, b):
        for tg in range(C // TG):
            means = []
            rstds = []
            ttfs = []

            def acc_g(j, carry):
                outs = []
                for k in range(TG):
                    t = tg * TG + k
                    x = gate_v[b, t, pl.ds(j * L, L)]
                    outs += [carry[2 * k] + x, carry[2 * k + 1] + x * x]
                return tuple(outs)

            def acc_q2(j, carry):
                outs = []
                for k in range(TG):
                    t = tg * TG + k
                    x = q2_v[b, t, pl.ds(j * L, L)]
                    outs += [carry[2 * k] + x, carry[2 * k + 1] + x * x]
                return tuple(outs)

            carr = (zero,) * (2 * TG)
            carr = plsc.parallel_loop(0, DG // L, unroll=4,
                                      carry=carr)(acc_g)
            carr = plsc.parallel_loop(0, DQ // L, unroll=4,
                                      carry=carr)(acc_q2)
            for k in range(TG):
                t = tg * TG + k
                ttv = tts_v[pl.ds(c * C + t, L)]
                ttf = _lane_shuffle(ttv.astype(jnp.float32), zidx)
                s = _allsum(carr[2 * k]) + s0v + ttf * dsv
                s2 = _allsum(carr[2 * k + 1]) + sq0v + ttf * dqv
                mean_v = s * (1.0 / H)
                var_v = s2 * (1.0 / H) - mean_v * mean_v
                means.append(mean_v)
                rstds.append(_rsqrt16(var_v + EPS))
                ttfs.append(ttf)

            def norm_g(j):
                g = gamma_v[pl.ds(j * L, L)]
                bta = beta_v[pl.ds(j * L, L)]
                for k in range(TG):
                    t = tg * TG + k
                    x = gate_v[b, t, pl.ds(j * L, L)]
                    out_v[b, t, pl.ds(j * L, L)] = \
                        (x - means[k]) * rstds[k] * g + bta

            def norm_q1(j):
                g = gamma_v[pl.ds(DG + j * L, L)]
                bta = beta_v[pl.ds(DG + j * L, L)]
                r0 = q01_v[0, pl.ds(j * L, L)]
                dd = diff_v[pl.ds(j * L, L)]
                for k in range(TG):
                    t = tg * TG + k
                    x = r0 + ttfs[k] * dd
                    out_v[b, t, pl.ds(DG + j * L, L)] = \
                        (x - means[k]) * rstds[k] * g + bta

            def norm_q2(j):
                g = gamma_v[pl.ds(DG + DQ + j * L, L)]
                bta = beta_v[pl.ds(DG + DQ + j * L, L)]
                for k in range(TG):
                    t = tg * TG + k
                    x = q2_v[b, t, pl.ds(j * L, L)]
                    out_v[b, t, pl.ds(DG + DQ + j * L, L)] = \
                        (x - means[k]) * rstds[k] * g + bta

            plsc.parallel_loop(0, DG // L, unroll=2)(norm_g)
            plsc.parallel_loop(0, DQ // L, unroll=2)(norm_q1)
            plsc.parallel_loop(0, DQ // L, unroll=2)(norm_q2)

    issue_gathers(0, 0)

    def pair_body(i, _):
        c0 = 2 * i

        issue_gathers(c0 + 1, 1)
        wait_gathers(0)

        @pl.when(i >= 1)
        def _():
            wait_out(0)

        compute(c0, 0)
        pltpu.async_copy(out_v.at[0], out_hbm.at[pl.ds(wbase + c0 * C, C)],
                         sem_out)

        @pl.when(i + 1 < NCH // 2)
        def _():
            issue_gathers(c0 + 2, 0)

        wait_gathers(1)

        @pl.when(i >= 1)
        def _():
            wait_out(1)

        compute(c0 + 1, 1)
        pltpu.async_copy(out_v.at[1],
                         out_hbm.at[pl.ds(wbase + (c0 + 1) * C, C)], sem_out)
        return 0

    lax.fori_loop(0, NCH // 2, pair_body, 0)
    wait_out(0)
    wait_out(1)


@jax.jit
def kernel(input_ids, token_type_ids, position_ids, gate_table, qubit_table,
           ln_gamma, ln_beta):
    ids = input_ids.astype(jnp.int32).reshape(N)
    tts = token_type_ids.astype(jnp.int32).reshape(N)
    pos = position_ids.astype(jnp.int32).reshape(N)

    mesh = plsc.VectorSubcoreMesh(core_axis_name="c", subcore_axis_name="s",
                                  num_cores=NC, num_subcores=NS)
    run = pl.kernel(
        _sc_body,
        out_type=jax.ShapeDtypeStruct((N, H), jnp.float32),
        mesh=mesh,
        scratch_types=[
            pltpu.VMEM((TOK,), jnp.int32),
            pltpu.VMEM((TOK + L,), jnp.int32),
            pltpu.VMEM((TOK,), jnp.int32),
            pltpu.VMEM((2, C, DG), jnp.float32),
            pltpu.VMEM((2, C, DQ), jnp.float32),
            pltpu.VMEM((2, C, H), jnp.float32),
            pltpu.VMEM((2, DQ), jnp.float32),
            pltpu.VMEM((DQ,), jnp.float32),
            pltpu.VMEM((H,), jnp.float32),
            pltpu.VMEM((H,), jnp.float32),
            pltpu.SemaphoreType.DMA,
            pltpu.SemaphoreType.DMA,
            pltpu.SemaphoreType.DMA,
        ],
    )
    out = run(ids, tts, pos, gate_table, qubit_table, ln_gamma, ln_beta)
    return out.reshape(B, S, H)
